# baseline (speedup 1.0000x reference)
import functools

import jax
import jax.numpy as jnp
from jax import lax
from jax.experimental import pallas as pl
from jax.experimental.pallas import tpu as pltpu
from jax.experimental.pallas import tpu_sc as plsc

N = 100000
B = 16384
D = 128
L = 16
NC, NS = 2, 16
NW = NC * NS
BPW = B // NW
CH = 64
NCH = BPW // CH
NSB = 4

_mesh = plsc.VectorSubcoreMesh(core_axis_name="c", subcore_axis_name="s")

_GATHER_DNUMS = lax.GatherDimensionNumbers(
    offset_dims=(), collapsed_slice_dims=(0,), start_index_map=(0,))


def _lane_broadcast(vec, r):
    idx = jnp.full((L, 1), r, jnp.int32)
    return lax.gather(vec, idx, _GATHER_DNUMS, slice_sizes=(1,),
                      mode=lax.GatherScatterMode.PROMISE_IN_BOUNDS)


@functools.partial(
    pl.kernel,
    mesh=_mesh,
    out_type=jax.ShapeDtypeStruct((B, D), jnp.float32),
    scratch_types=[
        pltpu.VMEM((NCH, CH), jnp.int32),
        pltpu.VMEM((NCH, CH), jnp.float32),
        pltpu.VMEM((D,), jnp.float32),
        pltpu.VMEM((D,), jnp.float32),
        pltpu.VMEM((NCH, CH, D), jnp.float32),
        pltpu.VMEM((NSB, CH, D), jnp.float32),
        pltpu.SemaphoreType.DMA((NCH,)),
        pltpu.SemaphoreType.DMA((NSB,)),
        pltpu.SemaphoreType.DMA,
    ],
)
def _tp_kernel(mem_hbm, dt_hbm, idx_hbm, w_hbm, b_hbm, out_hbm,
               idx_v, dt_v, w_v, b_v, rin_v, rout_v, gsem, ssem, psem):
    wid = lax.axis_index("s") * NC + lax.axis_index("c")
    base = wid * BPW

    pltpu.sync_copy(idx_hbm.at[pl.ds(wid * NCH, NCH)], idx_v)
    gathers = [
        pltpu.async_copy(mem_hbm.at[idx_v.at[c]], rin_v.at[c], gsem.at[c])
        for c in range(NCH)
    ]
    dt_cp = pltpu.async_copy(dt_hbm.at[pl.ds(wid * NCH, NCH)], dt_v, psem)
    pltpu.sync_copy(w_hbm, w_v)
    pltpu.sync_copy(b_hbm, b_v)
    dt_cp.wait()

    wreg = [w_v[pl.ds(L * j, L)] for j in range(D // L)]
    breg = [b_v[pl.ds(L * j, L)] + 1.0 for j in range(D // L)]

    scatters = [None] * NSB
    for c in range(NCH):
        p = c % NSB
        gathers[c].wait()
        if scatters[p] is not None:
            scatters[p].wait()

        def _block(blk, carry, c=c, p=p):
            dtv = dt_v[c, pl.ds(blk * L, L)]
            for r in range(L):
                dtb = _lane_broadcast(dtv, r)
                i = blk * L + r
                for j in range(D // L):
                    sl = pl.ds(L * j, L)
                    rout_v[p, i, sl] = rin_v[c, i, sl] * (dtb * wreg[j] + breg[j])
            return carry

        lax.fori_loop(0, CH // L, _block, 0)
        scatters[p] = pltpu.async_copy(
            rout_v.at[p], out_hbm.at[pl.ds(base + c * CH, CH)], ssem.at[p])

    for p in range(NSB):
        scatters[p].wait()


def kernel(memory, delta_t, tar_idx, W, b):
    idx = tar_idx.reshape(B // CH, CH).astype(jnp.int32)
    dt = delta_t.reshape(B // CH, CH)
    w = W.reshape(D)
    return _tp_kernel(memory, dt, idx, w, b)


# speedup vs baseline: 1.0915x; 1.0915x over previous
import functools

import jax
import jax.numpy as jnp
from jax import lax
from jax.experimental import pallas as pl
from jax.experimental.pallas import tpu as pltpu
from jax.experimental.pallas import tpu_sc as plsc

N = 100000
B = 16384
D = 128
L = 16
NC, NS = 2, 16
NW = NC * NS
BPW = B // NW
CH = 128
NCH = BPW // CH
NSB = 2

_mesh = plsc.VectorSubcoreMesh(core_axis_name="c", subcore_axis_name="s")

_GATHER_DNUMS = lax.GatherDimensionNumbers(
    offset_dims=(), collapsed_slice_dims=(0,), start_index_map=(0,))


def _lane_broadcast(vec, r):
    idx = jnp.full((L, 1), r, jnp.int32)
    return lax.gather(vec, idx, _GATHER_DNUMS, slice_sizes=(1,),
                      mode=lax.GatherScatterMode.PROMISE_IN_BOUNDS)


@functools.partial(
    pl.kernel,
    mesh=_mesh,
    out_type=jax.ShapeDtypeStruct((B, D), jnp.float32),
    scratch_types=[
        pltpu.VMEM((NCH, CH), jnp.int32),
        pltpu.VMEM((NCH, CH), jnp.float32),
        pltpu.VMEM((D,), jnp.float32),
        pltpu.VMEM((D,), jnp.float32),
        pltpu.VMEM((NCH, CH, D), jnp.float32),
        pltpu.VMEM((NSB, CH, D), jnp.float32),
        pltpu.SemaphoreType.DMA((NCH,)),
        pltpu.SemaphoreType.DMA((NSB,)),
        pltpu.SemaphoreType.DMA,
    ],
)
def _tp_kernel(mem_hbm, dt_hbm, idx_hbm, w_hbm, b_hbm, out_hbm,
               idx_v, dt_v, w_v, b_v, rin_v, rout_v, gsem, ssem, psem):
    wid = lax.axis_index("s") * NC + lax.axis_index("c")
    base = wid * BPW

    pltpu.sync_copy(idx_hbm.at[pl.ds(wid * NCH, NCH)], idx_v)
    gathers = [
        pltpu.async_copy(mem_hbm.at[idx_v.at[c]], rin_v.at[c], gsem.at[c])
        for c in range(NCH)
    ]
    dt_cp = pltpu.async_copy(dt_hbm.at[pl.ds(wid * NCH, NCH)], dt_v, psem)
    pltpu.sync_copy(w_hbm, w_v)
    pltpu.sync_copy(b_hbm, b_v)
    dt_cp.wait()

    wreg = [w_v[pl.ds(L * j, L)] for j in range(D // L)]
    breg = [b_v[pl.ds(L * j, L)] + 1.0 for j in range(D // L)]

    scatters = [None] * NSB
    for c in range(NCH):
        p = c % NSB
        gathers[c].wait()
        if scatters[p] is not None:
            scatters[p].wait()

        def _block(blk, carry, c=c, p=p):
            dtv = dt_v[c, pl.ds(blk * L, L)]
            for r in range(L):
                dtb = _lane_broadcast(dtv, r)
                i = blk * L + r
                for j in range(D // L):
                    sl = pl.ds(L * j, L)
                    rout_v[p, i, sl] = rin_v[c, i, sl] * (dtb * wreg[j] + breg[j])
            return carry

        lax.fori_loop(0, CH // L, _block, 0)
        scatters[p] = pltpu.async_copy(
            rout_v.at[p], out_hbm.at[pl.ds(base + c * CH, CH)], ssem.at[p])

    for p in range(NSB):
        scatters[p].wait()


def kernel(memory, delta_t, tar_idx, W, b):
    idx = tar_idx.reshape(B // CH, CH).astype(jnp.int32)
    dt = delta_t.reshape(B // CH, CH)
    w = W.reshape(D)
    return _tp_kernel(memory, dt, idx, w, b)
